# baseline (speedup 1.0000x reference)
import functools

import jax
import jax.numpy as jnp
from jax import lax
from jax.experimental import pallas as pl
from jax.experimental.pallas import tpu as pltpu
from jax.experimental.pallas import tpu_sc as plsc

N_STATES = 32 * 32 * 32 * 32
N_ACTIONS = 16
BATCH = 16384

NC = 2
NS = 16
L = 16
NW = NC * NS
B_PER_W = BATCH // NW
E_PER_W = B_PER_W * N_ACTIONS

_mesh = plsc.VectorSubcoreMesh(core_axis_name="c", subcore_axis_name="s")


@functools.partial(
    pl.kernel,
    out_type=jax.ShapeDtypeStruct((BATCH * N_ACTIONS,), jnp.float32),
    mesh=_mesh,
    scratch_types=[
        pltpu.VMEM((B_PER_W,), jnp.int32),
        pltpu.VMEM((E_PER_W,), jnp.int32),
        pltpu.VMEM((E_PER_W,), jnp.float32),
        pltpu.SemaphoreType.DMA,
    ],
)
def _qlookup(w_hbm, obs_hbm, out_hbm, obs_v, eidx_v, rows_v, sem):
    wid = lax.axis_index("s") * NC + lax.axis_index("c")
    base = wid * B_PER_W
    pltpu.sync_copy(obs_hbm.at[pl.ds(base, B_PER_W)], obs_v)

    action_off = lax.iota(jnp.int32, L) * N_STATES
    _dnums = lax.GatherDimensionNumbers(
        offset_dims=(), collapsed_slice_dims=(0,), start_index_map=(0,))

    def _bcast_lane(vec, i):
        idx = jnp.full((L, 1), i, jnp.int32)
        return lax.gather(vec, idx, _dnums, slice_sizes=(1,),
                          mode=lax.GatherScatterMode.PROMISE_IN_BOUNDS)

    def build(j, carry):
        o = obs_v[pl.ds(j * L, L)]
        for i in range(L):
            b = j * L + i
            eidx_v[pl.ds(b * N_ACTIONS, N_ACTIONS)] = _bcast_lane(o, i) + action_off
        return carry

    lax.fori_loop(0, B_PER_W // L, build, 0)

    pltpu.async_copy(w_hbm.at[eidx_v], rows_v, sem).wait()
    pltpu.sync_copy(rows_v, out_hbm.at[pl.ds(base * N_ACTIONS, E_PER_W)])


def kernel(v_obs, W):
    out = _qlookup(W.reshape(-1), v_obs.astype(jnp.int32))
    return out.reshape(BATCH, N_ACTIONS)


# speedup vs baseline: 2.1732x; 2.1732x over previous
import functools

import jax
import jax.numpy as jnp
from jax import lax
from jax.experimental import pallas as pl
from jax.experimental.pallas import tpu as pltpu
from jax.experimental.pallas import tpu_sc as plsc

N_STATES = 32 * 32 * 32 * 32
N_ACTIONS = 16
BATCH = 16384

NC = 2
NS = 16
L = 16
NW = NC * NS
B_PER_W = BATCH // NW
E_PER_W = B_PER_W * N_ACTIONS

_mesh = plsc.VectorSubcoreMesh(core_axis_name="c", subcore_axis_name="s")


@functools.partial(
    pl.kernel,
    out_type=jax.ShapeDtypeStruct((BATCH * N_ACTIONS,), jnp.float32),
    mesh=_mesh,
    scratch_types=[
        pltpu.VMEM((B_PER_W,), jnp.int32),
        pltpu.VMEM((E_PER_W,), jnp.int32),
        pltpu.VMEM((E_PER_W,), jnp.float32),
        pltpu.SemaphoreType.DMA,
    ],
)
def _qlookup(w_hbm, obs_hbm, out_hbm, obs_v, eidx_v, rows_v, sem):
    wid = lax.axis_index("s") * NC + lax.axis_index("c")
    base = wid * B_PER_W
    pltpu.sync_copy(obs_hbm.at[pl.ds(base, B_PER_W)], obs_v)

    a_iota = lax.iota(jnp.int32, L)
    action_off = (a_iota >> 3) * (8 * N_STATES) + (a_iota & 7) * 128
    _dnums = lax.GatherDimensionNumbers(
        offset_dims=(), collapsed_slice_dims=(0,), start_index_map=(0,))

    def _bcast_lane(vec, i):
        idx = jnp.full((L, 1), i, jnp.int32)
        return lax.gather(vec, idx, _dnums, slice_sizes=(1,),
                          mode=lax.GatherScatterMode.PROMISE_IN_BOUNDS)

    def build(j, carry):
        o = obs_v[pl.ds(j * L, L)]
        s_off = ((o >> 7) << 10) + (o & 127)
        for i in range(L):
            b = j * L + i
            eidx_v[pl.ds(b * N_ACTIONS, N_ACTIONS)] = (
                _bcast_lane(s_off, i) + action_off)
        return carry

    lax.fori_loop(0, B_PER_W // L, build, 0)

    pltpu.async_copy(w_hbm.at[eidx_v], rows_v, sem).wait()
    pltpu.sync_copy(rows_v, out_hbm.at[pl.ds(base * N_ACTIONS, E_PER_W)])


def kernel(v_obs, W):
    w_flat = (W.reshape(2, 8, 8192, 128)
               .transpose(0, 2, 1, 3)
               .reshape(N_ACTIONS * N_STATES))
    out = _qlookup(w_flat, v_obs.astype(jnp.int32))
    return out.reshape(BATCH, N_ACTIONS)


# speedup vs baseline: 2.9373x; 1.3516x over previous
import functools

import jax
import jax.numpy as jnp
from jax import lax
from jax.experimental import pallas as pl
from jax.experimental.pallas import tpu as pltpu
from jax.experimental.pallas import tpu_sc as plsc

N_STATES = 32 * 32 * 32 * 32
N_ACTIONS = 16
BATCH = 16384

NC = 2
NS = 16
L = 16
NW = NC * NS
B_PER_W = BATCH // NW
E_PER_W = B_PER_W * N_ACTIONS

_ACTION_OFF = [(a >> 3) * (8 * N_STATES) + (a & 7) * 128
               for a in range(N_ACTIONS)]

_mesh = plsc.VectorSubcoreMesh(core_axis_name="c", subcore_axis_name="s")


@functools.partial(
    pl.kernel,
    out_type=jax.ShapeDtypeStruct((N_ACTIONS, BATCH), jnp.float32),
    mesh=_mesh,
    scratch_types=[
        pltpu.VMEM((B_PER_W,), jnp.int32),
        pltpu.VMEM((E_PER_W,), jnp.int32),
        pltpu.VMEM((E_PER_W,), jnp.float32),
        pltpu.SemaphoreType.DMA,
    ],
)
def _qlookup(w_hbm, obs_hbm, out_hbm, obs_v, eidx_v, rows_v, sem):
    wid = lax.axis_index("s") * NC + lax.axis_index("c")
    base = wid * B_PER_W
    pltpu.sync_copy(obs_hbm.at[pl.ds(base, B_PER_W)], obs_v)

    def build(j, carry):
        o = obs_v[pl.ds(j * L, L)]
        s_off = ((o >> 7) << 10) + (o & 127)
        for a in range(N_ACTIONS):
            eidx_v[pl.ds(a * B_PER_W + j * L, L)] = s_off + _ACTION_OFF[a]
        return carry

    lax.fori_loop(0, B_PER_W // L, build, 0)

    pltpu.async_copy(w_hbm.at[eidx_v], rows_v, sem).wait()

    for a in range(N_ACTIONS):
        pltpu.sync_copy(rows_v.at[pl.ds(a * B_PER_W, B_PER_W)],
                        out_hbm.at[a, pl.ds(base, B_PER_W)])


def kernel(v_obs, W):
    w_flat = (W.reshape(2, 8, 8192, 128)
               .transpose(0, 2, 1, 3)
               .reshape(N_ACTIONS * N_STATES))
    out = _qlookup(w_flat, v_obs.astype(jnp.int32))
    return out.T
